# Initial kernel scaffold; baseline (speedup 1.0000x reference)
import functools

import jax
import jax.numpy as jnp
from jax import lax
from jax.experimental import pallas as pl
from jax.experimental.pallas import tpu as pltpu
from jax.experimental.pallas import tpu_sc as plsc

B = 4096
S = 200
T = B * S
HD = 64
D = 128
NC = 2
NS = 16
NW = NC * NS
TPW = T // NW
CH = 256
NCH = TPW // CH
IR = CH // 128

_mesh = plsc.VectorSubcoreMesh(core_axis_name="c", subcore_axis_name="s")


@functools.partial(
    pl.kernel,
    out_type=jax.ShapeDtypeStruct((T, D), jnp.float32),
    mesh=_mesh,
    scratch_types=[
        pltpu.VMEM((IR, 128), jnp.int32),
        pltpu.VMEM((IR, 128), jnp.int32),
        pltpu.VMEM((CH,), jnp.float32),
        pltpu.VMEM((CH, HD), jnp.float32),
        pltpu.VMEM((CH, HD), jnp.float32),
        pltpu.VMEM((CH, D), jnp.float32),
        pltpu.SemaphoreType.DMA,
    ],
)
def _pe_kernel(depth_hbm, pos_hbm, mask_hbm, dpe_hbm, ppe_hbm, out_hbm,
               idx_d, idx_p, mask_v, d_rows, p_rows, obuf, sem):
    wid = lax.axis_index("s") * NC + lax.axis_index("c")

    def chunk_body(c, carry):
        base = pl.multiple_of(wid * TPW + c * CH, CH)
        row_base = pl.multiple_of(wid * (TPW // 128) + c * IR, IR)

        pltpu.sync_copy(depth_hbm.at[pl.ds(row_base, IR)], idx_d)
        pltpu.sync_copy(pos_hbm.at[pl.ds(row_base, IR)], idx_p)
        pltpu.sync_copy(mask_hbm.at[pl.ds(base, CH)], mask_v)

        copies = []
        for r in range(IR):
            copies.append(pltpu.async_copy(
                dpe_hbm.at[idx_d.at[r]],
                d_rows.at[pl.ds(r * 128, 128)], sem))
            copies.append(pltpu.async_copy(
                ppe_hbm.at[idx_p.at[r]],
                p_rows.at[pl.ds(r * 128, 128)], sem))
        for cp in copies:
            cp.wait()

        def tok_body(t, tc):
            m = mask_v[t]
            for j in range(4):
                obuf[t, pl.ds(16 * j, 16)] = d_rows[t, pl.ds(16 * j, 16)] * m
            for j in range(4):
                obuf[t, pl.ds(HD + 16 * j, 16)] = p_rows[t, pl.ds(16 * j, 16)] * m
            return tc

        lax.fori_loop(0, CH, tok_body, 0)

        pltpu.sync_copy(obuf, out_hbm.at[pl.ds(base, CH)])
        return carry

    lax.fori_loop(0, NCH, chunk_body, 0)


def kernel(depth, position, mask, depth_pe, position_pe):
    depth_r = depth.reshape(T // 128, 128)
    pos_r = position.reshape(T // 128, 128)
    mask_f = mask.reshape(T)
    out = _pe_kernel(depth_r, pos_r, mask_f, depth_pe, position_pe)
    return out.reshape(B, S, D)


# speedup vs baseline: 6.1489x; 6.1489x over previous
import functools

import jax
import jax.numpy as jnp
from jax import lax
from jax.experimental import pallas as pl
from jax.experimental.pallas import tpu as pltpu
from jax.experimental.pallas import tpu_sc as plsc

B = 4096
S = 200
T = B * S
HD = 64
D = 128
NC = 2
NS = 16
NW = NC * NS
TPW = T // NW
CH = 256
IR = CH // 128
NCH = TPW // CH

_mesh = plsc.VectorSubcoreMesh(core_axis_name="c", subcore_axis_name="s")


@functools.partial(
    pl.kernel,
    out_type=jax.ShapeDtypeStruct((T, D), jnp.float32),
    mesh=_mesh,
    scratch_types=[
        pltpu.VMEM((IR, 128), jnp.int32),
        pltpu.VMEM((IR, 128), jnp.int32),
        pltpu.VMEM((CH,), jnp.float32),
        pltpu.VMEM((CH, D), jnp.float32),
        pltpu.VMEM((CH, D), jnp.float32),
        pltpu.VMEM((CH, D), jnp.float32),
        pltpu.SemaphoreType.DMA,
    ],
)
def _pe_kernel(depth_hbm, pos_hbm, mask_hbm, dpe_hbm, ppe_hbm,
               out_hbm, idx_d, idx_p, mask_v, d_rows, p_rows, obuf, sem):
    wid = lax.axis_index("s") * NC + lax.axis_index("c")

    def chunk_body(c, carry):
        base = pl.multiple_of(wid * TPW + c * CH, CH)
        for rr in range(IR):
            pltpu.sync_copy(depth_hbm.at[pl.ds(base + rr * 128, 128)],
                            idx_d.at[rr])
            pltpu.sync_copy(pos_hbm.at[pl.ds(base + rr * 128, 128)],
                            idx_p.at[rr])
        pltpu.sync_copy(mask_hbm.at[pl.ds(base, CH)], mask_v)

        copies = []
        for rr in range(IR):
            copies.append(pltpu.async_copy(
                dpe_hbm.at[idx_d.at[rr]],
                d_rows.at[pl.ds(rr * 128, 128)], sem))
            copies.append(pltpu.async_copy(
                ppe_hbm.at[idx_p.at[rr]],
                p_rows.at[pl.ds(rr * 128, 128)], sem))
        for cp in copies:
            cp.wait()

        def grp_body(g, gc):
            t0 = g * 16
            mask16 = mask_v[pl.ds(t0, 16)]
            for i in range(16):
                m = mask16[i]
                t = t0 + i
                for j in range(4):
                    obuf[t, pl.ds(16 * j, 16)] = (
                        d_rows[t, pl.ds(16 * j, 16)] * m)
                for j in range(4, 8):
                    obuf[t, pl.ds(16 * j, 16)] = (
                        p_rows[t, pl.ds(16 * j, 16)] * m)
            return gc

        lax.fori_loop(0, CH // 16, grp_body, 0)

        pltpu.sync_copy(obuf, out_hbm.at[pl.ds(base, CH)])
        return carry

    lax.fori_loop(0, NCH, chunk_body, 0)


def kernel(depth, position, mask, depth_pe, position_pe):
    depth_f = depth.reshape(T)
    pos_f = position.reshape(T)
    mask_f = mask.reshape(T)
    dpe_pad = jnp.pad(depth_pe, ((0, 0), (0, HD)))
    ppe_pad = jnp.pad(position_pe, ((0, 0), (HD, 0)))
    out = _pe_kernel(depth_f, pos_f, mask_f, dpe_pad, ppe_pad)
    return out.reshape(B, S, D)


# speedup vs baseline: 14.7419x; 2.3975x over previous
import functools

import jax
import jax.numpy as jnp
from jax import lax
from jax.experimental import pallas as pl
from jax.experimental.pallas import tpu as pltpu
from jax.experimental.pallas import tpu_sc as plsc

B = 4096
S = 200
T = B * S
HD = 64
ND = 512
NP = 2048
D = 128
NC = 2
NS = 16
NW = NC * NS
TPW = T // NW
CH = 128
NCH = TPW // CH

_mesh = plsc.VectorSubcoreMesh(core_axis_name="c", subcore_axis_name="s")


@functools.partial(
    pl.kernel,
    out_type=jax.ShapeDtypeStruct((T, D), jnp.float32),
    mesh=_mesh,
    scratch_types=[
        pltpu.VMEM((2, 1, 128), jnp.int32),
        pltpu.VMEM((2, 1, 128), jnp.int32),
        pltpu.VMEM((2, CH), jnp.float32),
        pltpu.VMEM((2, CH, D), jnp.float32),
        pltpu.VMEM((2, CH, D), jnp.float32),
        pltpu.VMEM((2, CH, D), jnp.float32),
        pltpu.VMEM_SHARED((ND, D), jnp.float32),
        pltpu.VMEM_SHARED((NP, D), jnp.float32),
        pltpu.SemaphoreType.DMA((2,)),
        pltpu.SemaphoreType.DMA((2,)),
        pltpu.SemaphoreType.DMA((2,)),
    ],
)
def _pe_kernel(depth_hbm, pos_hbm, mask_hbm, dpe_hbm, ppe_hbm,
               out_hbm, idx_d, idx_p, mask_v, d_rows, p_rows, obuf,
               dpe_s, ppe_s, sem_in, sem_g, sem_out):
    wid = lax.axis_index("s") * NC + lax.axis_index("c")

    @pl.when(lax.axis_index("s") == 0)
    def _():
        pltpu.sync_copy(dpe_hbm, dpe_s)
        pltpu.sync_copy(ppe_hbm, ppe_s)

    plsc.subcore_barrier()

    def tbase(c):
        return pl.multiple_of(wid * TPW + c * CH, CH)

    def start_in(c, k):
        b = tbase(c)
        pltpu.async_copy(depth_hbm.at[pl.ds(b, CH)], idx_d.at[k, 0],
                         sem_in.at[k])
        pltpu.async_copy(pos_hbm.at[pl.ds(b, CH)], idx_p.at[k, 0],
                         sem_in.at[k])
        pltpu.async_copy(mask_hbm.at[pl.ds(b, CH)], mask_v.at[k],
                         sem_in.at[k])

    def wait_in(k):
        pltpu.make_async_copy(depth_hbm.at[pl.ds(0, CH)], idx_d.at[k, 0],
                              sem_in.at[k]).wait()
        pltpu.make_async_copy(pos_hbm.at[pl.ds(0, CH)], idx_p.at[k, 0],
                              sem_in.at[k]).wait()
        pltpu.make_async_copy(mask_hbm.at[pl.ds(0, CH)], mask_v.at[k],
                              sem_in.at[k]).wait()

    def start_gather(k):
        pltpu.async_copy(dpe_s.at[idx_d.at[k, 0]], d_rows.at[k], sem_g.at[k])
        pltpu.async_copy(ppe_s.at[idx_p.at[k, 0]], p_rows.at[k], sem_g.at[k])

    def wait_gather(k):
        pltpu.make_async_copy(dpe_s.at[idx_d.at[k, 0]], d_rows.at[k],
                              sem_g.at[k]).wait()
        pltpu.make_async_copy(ppe_s.at[idx_p.at[k, 0]], p_rows.at[k],
                              sem_g.at[k]).wait()

    def start_out(c, k):
        pltpu.async_copy(obuf.at[k], out_hbm.at[pl.ds(tbase(c), CH)],
                         sem_out.at[k])

    def wait_out(k):
        pltpu.make_async_copy(obuf.at[k], out_hbm.at[pl.ds(0, CH)],
                              sem_out.at[k]).wait()

    def compute(k):
        def grp_body(g, gc):
            t0 = g * 16
            mask16 = mask_v[k, pl.ds(t0, 16)]
            for i in range(16):
                m = mask16[i]
                t = t0 + i
                for j in range(4):
                    obuf[k, t, pl.ds(16 * j, 16)] = (
                        d_rows[k, t, pl.ds(16 * j, 16)] * m)
                for j in range(4, 8):
                    obuf[k, t, pl.ds(16 * j, 16)] = (
                        p_rows[k, t, pl.ds(16 * j, 16)] * m)
            return gc

        lax.fori_loop(0, CH // 16, grp_body, 0)

    start_in(0, 0)
    wait_in(0)
    start_gather(0)
    start_in(1, 1)

    def loop_body(c2, carry):
        for k in (0, 1):
            c = c2 * 2 + k

            @pl.when(c + 1 < NCH)
            def _():
                wait_in(k ^ 1)
                start_gather(k ^ 1)

            wait_gather(k)

            @pl.when(c >= 2)
            def _():
                wait_out(k)

            compute(k)
            start_out(c, k)

            @pl.when(c + 2 < NCH)
            def _():
                start_in(c + 2, k)
        return carry

    lax.fori_loop(0, NCH // 2, loop_body, 0)
    wait_out(0)
    wait_out(1)


def kernel(depth, position, mask, depth_pe, position_pe):
    depth_f = depth.reshape(T)
    pos_f = position.reshape(T)
    mask_f = mask.reshape(T)
    dpe_pad = jnp.pad(depth_pe, ((0, 0), (0, HD)))
    ppe_pad = jnp.pad(position_pe, ((0, 0), (HD, 0)))
    out = _pe_kernel(depth_f, pos_f, mask_f, dpe_pad, ppe_pad)
    return out.reshape(B, S, D)


# speedup vs baseline: 14.8175x; 1.0051x over previous
import functools

import jax
import jax.numpy as jnp
from jax import lax
from jax.experimental import pallas as pl
from jax.experimental.pallas import tpu as pltpu
from jax.experimental.pallas import tpu_sc as plsc

B = 4096
S = 200
T = B * S
HD = 64
ND = 512
NP = 2048
D = 128
NC = 2
NS = 16
NW = NC * NS
TPW = T // NW
CH = 128
NCH = TPW // CH

_mesh = plsc.VectorSubcoreMesh(core_axis_name="c", subcore_axis_name="s")


@functools.partial(
    pl.kernel,
    out_type=jax.ShapeDtypeStruct((T, D), jnp.float32),
    mesh=_mesh,
    scratch_types=[
        pltpu.VMEM((2, 1, 128), jnp.int32),
        pltpu.VMEM((2, 1, 128), jnp.int32),
        pltpu.VMEM((2, CH), jnp.float32),
        pltpu.VMEM((2, CH, HD), jnp.float32),
        pltpu.VMEM((2, CH, HD), jnp.float32),
        pltpu.VMEM((2, CH, D), jnp.float32),
        pltpu.VMEM_SHARED((ND, HD), jnp.float32),
        pltpu.VMEM_SHARED((NP, HD), jnp.float32),
        pltpu.SemaphoreType.DMA((2,)),
        pltpu.SemaphoreType.DMA((2,)),
        pltpu.SemaphoreType.DMA((2,)),
    ],
)
def _pe_kernel(depth_hbm, pos_hbm, mask_hbm, dpe_hbm, ppe_hbm,
               out_hbm, idx_d, idx_p, mask_v, d_rows, p_rows, obuf,
               dpe_s, ppe_s, sem_in, sem_g, sem_out):
    wid = lax.axis_index("s") * NC + lax.axis_index("c")

    @pl.when(lax.axis_index("s") == 0)
    def _():
        pltpu.sync_copy(dpe_hbm, dpe_s)
        pltpu.sync_copy(ppe_hbm, ppe_s)

    plsc.subcore_barrier()

    def tbase(c):
        return pl.multiple_of(wid * TPW + c * CH, CH)

    def start_in(c, k):
        b = tbase(c)
        pltpu.async_copy(depth_hbm.at[pl.ds(b, CH)], idx_d.at[k, 0],
                         sem_in.at[k])
        pltpu.async_copy(pos_hbm.at[pl.ds(b, CH)], idx_p.at[k, 0],
                         sem_in.at[k])
        pltpu.async_copy(mask_hbm.at[pl.ds(b, CH)], mask_v.at[k],
                         sem_in.at[k])

    def wait_in(k):
        pltpu.make_async_copy(depth_hbm.at[pl.ds(0, CH)], idx_d.at[k, 0],
                              sem_in.at[k]).wait()
        pltpu.make_async_copy(pos_hbm.at[pl.ds(0, CH)], idx_p.at[k, 0],
                              sem_in.at[k]).wait()
        pltpu.make_async_copy(mask_hbm.at[pl.ds(0, CH)], mask_v.at[k],
                              sem_in.at[k]).wait()

    def start_gather(k):
        pltpu.async_copy(dpe_s.at[idx_d.at[k, 0]], d_rows.at[k], sem_g.at[k])
        pltpu.async_copy(ppe_s.at[idx_p.at[k, 0]], p_rows.at[k], sem_g.at[k])

    def wait_gather(k):
        pltpu.make_async_copy(dpe_s.at[idx_d.at[k, 0]], d_rows.at[k],
                              sem_g.at[k]).wait()
        pltpu.make_async_copy(ppe_s.at[idx_p.at[k, 0]], p_rows.at[k],
                              sem_g.at[k]).wait()

    def start_out(c, k):
        pltpu.async_copy(obuf.at[k], out_hbm.at[pl.ds(tbase(c), CH)],
                         sem_out.at[k])

    def wait_out(k):
        pltpu.make_async_copy(obuf.at[k], out_hbm.at[pl.ds(0, CH)],
                              sem_out.at[k]).wait()

    def compute(k):
        def grp_body(g, gc):
            t0 = g * 16
            mask16 = mask_v[k, pl.ds(t0, 16)]
            for i in range(16):
                m = mask16[i]
                t = t0 + i
                for j in range(4):
                    obuf[k, t, pl.ds(16 * j, 16)] = (
                        d_rows[k, t, pl.ds(16 * j, 16)] * m)
                for j in range(4):
                    obuf[k, t, pl.ds(HD + 16 * j, 16)] = (
                        p_rows[k, t, pl.ds(16 * j, 16)] * m)
            return gc

        lax.fori_loop(0, CH // 16, grp_body, 0)

    start_in(0, 0)
    wait_in(0)
    start_gather(0)
    start_in(1, 1)

    def loop_body(c2, carry):
        for k in (0, 1):
            c = c2 * 2 + k

            @pl.when(c + 1 < NCH)
            def _():
                wait_in(k ^ 1)
                start_gather(k ^ 1)

            wait_gather(k)

            @pl.when(c >= 2)
            def _():
                wait_out(k)

            compute(k)
            start_out(c, k)

            @pl.when(c + 2 < NCH)
            def _():
                start_in(c + 2, k)
        return carry

    lax.fori_loop(0, NCH // 2, loop_body, 0)
    wait_out(0)
    wait_out(1)


def kernel(depth, position, mask, depth_pe, position_pe):
    depth_f = depth.reshape(T)
    pos_f = position.reshape(T)
    mask_f = mask.reshape(T)
    out = _pe_kernel(depth_f, pos_f, mask_f, depth_pe, position_pe)
    return out.reshape(B, S, D)
